# baseline (speedup 1.0000x reference)
import jax
import jax.numpy as jnp
from jax import lax
from jax.experimental import pallas as pl
from jax.experimental.pallas import tpu as pltpu
from jax.experimental.pallas import tpu_sc as plsc

K = 32
B = 16384

NC = 2
NS = 16
NW = NC * NS
BPW = B // NW
NCHUNK = BPW // 128
NGROUP = BPW // 16


def _gmf_body(x_hbm, wb_hbm, user_hbm, item_hbm, out_hbm,
              xv, uidx, iidx, urows, irows, wbv, outv, sem):
    wid = lax.axis_index("s") * NC + lax.axis_index("c")
    base = wid * BPW

    pltpu.sync_copy(x_hbm.at[pl.ds(base, BPW)], xv)
    pltpu.sync_copy(wb_hbm, wbv)

    iota16 = lax.iota(jnp.int32, 16)
    zeros16 = jnp.zeros((16,), jnp.int32)
    ones16 = jnp.ones((16,), jnp.int32)

    for j in range(NGROUP):
        rows = iota16 + (j * 16)
        chunk, off = (j * 16) // 128, (j * 16) % 128
        uidx[chunk, pl.ds(off, 16)] = plsc.load_gather(xv, [rows, zeros16])
        iidx[chunk, pl.ds(off, 16)] = plsc.load_gather(xv, [rows, ones16])

    copies = []
    for c in range(NCHUNK):
        copies.append(pltpu.async_copy(
            user_hbm.at[uidx.at[c]], urows.at[pl.ds(c * 128, 128)], sem))
        copies.append(pltpu.async_copy(
            item_hbm.at[iidx.at[c]], irows.at[pl.ds(c * 128, 128)], sem))
    for cp in copies:
        cp.wait()

    w0 = wbv[pl.ds(0, 16)]
    w1 = wbv[pl.ds(16, 16)]
    bias = wbv[pl.ds(24, 16)][8]

    def group(g, carry):
        rows = iota16 + g * 16
        acc = jnp.zeros((16,), jnp.float32)
        for k in range(K):
            kvec = jnp.full((16,), k, jnp.int32)
            ucol = plsc.load_gather(urows, [rows, kvec])
            icol = plsc.load_gather(irows, [rows, kvec])
            wk = w0[k] if k < 16 else w1[k - 16]
            acc = acc + ucol * icol * wk
        z = acc + bias
        outv[pl.ds(g * 16, 16)] = 1.0 / (1.0 + jnp.exp(-z))
        return carry

    lax.fori_loop(0, NGROUP, group, None)

    pltpu.sync_copy(outv, out_hbm.at[pl.ds(base, BPW)])


@jax.jit
def kernel(x, user_table, item_table, W, b):
    wb = jnp.concatenate([W.reshape(K), jnp.pad(b, (0, 7))]).astype(jnp.float32)
    mesh = plsc.VectorSubcoreMesh(core_axis_name="c", subcore_axis_name="s")
    out = pl.kernel(
        _gmf_body,
        out_type=jax.ShapeDtypeStruct((B,), jnp.float32),
        mesh=mesh,
        compiler_params=pltpu.CompilerParams(
            needs_layout_passes=False, use_tc_tiling_on_sc=False),
        scratch_types=[
            pltpu.VMEM((BPW, 2), jnp.int32),
            pltpu.VMEM((NCHUNK, 128), jnp.int32),
            pltpu.VMEM((NCHUNK, 128), jnp.int32),
            pltpu.VMEM((BPW, K), jnp.float32),
            pltpu.VMEM((BPW, K), jnp.float32),
            pltpu.VMEM((40,), jnp.float32),
            pltpu.VMEM((BPW,), jnp.float32),
            pltpu.SemaphoreType.DMA,
        ],
    )(x.astype(jnp.int32), wb, user_table, item_table)
    return out.reshape(B, 1, 1)


# speedup vs baseline: 1.0132x; 1.0132x over previous
import jax
import jax.numpy as jnp
from jax import lax
from jax.experimental import pallas as pl
from jax.experimental.pallas import tpu as pltpu
from jax.experimental.pallas import tpu_sc as plsc

K = 32
B = 16384

NC = 2
NS = 16
NW = NC * NS
BPW = B // NW
NCHUNK = BPW // 128
NGROUP = BPW // 16


def _gmf_body(x_hbm, wb_hbm, user_hbm, item_hbm, out_hbm,
              xv, uidx_lo, uidx_hi, iidx_lo, iidx_hi,
              ulo, uhi, ilo, ihi, wbv, outv, sem):
    wid = lax.axis_index("s") * NC + lax.axis_index("c")
    base = wid * BPW

    pltpu.sync_copy(x_hbm.at[pl.ds(base * 2, 2 * BPW)], xv)
    pltpu.sync_copy(wb_hbm, wbv)

    iota16 = lax.iota(jnp.int32, 16)

    for j in range(NGROUP):
        chunk, off = (j * 16) // 128, (j * 16) % 128
        ucol = plsc.load_gather(xv, [iota16 * 2 + (j * 32)])
        icol = plsc.load_gather(xv, [iota16 * 2 + (j * 32 + 1)])
        uidx_lo[chunk, pl.ds(off, 16)] = ucol * 2
        uidx_hi[chunk, pl.ds(off, 16)] = ucol * 2 + 1
        iidx_lo[chunk, pl.ds(off, 16)] = icol * 2
        iidx_hi[chunk, pl.ds(off, 16)] = icol * 2 + 1

    copies = []
    for c in range(NCHUNK):
        for idxref, table, dst in ((uidx_lo, user_hbm, ulo),
                                   (uidx_hi, user_hbm, uhi),
                                   (iidx_lo, item_hbm, ilo),
                                   (iidx_hi, item_hbm, ihi)):
            copies.append(pltpu.async_copy(
                table.at[idxref.at[c]], dst.at[pl.ds(c * 128, 128)], sem))
    for cp in copies:
        cp.wait()

    w0 = wbv[pl.ds(0, 16)]
    w1 = wbv[pl.ds(16, 16)]
    bias = wbv[pl.ds(24, 16)][8]

    def group(g, carry):
        rows = iota16 + g * 16
        acc = jnp.zeros((16,), jnp.float32)
        for k in range(K):
            uref = ulo if k < 16 else uhi
            iref = ilo if k < 16 else ihi
            kvec = jnp.full((16,), k % 16, jnp.int32)
            ucol = plsc.load_gather(uref, [rows, kvec])
            icol = plsc.load_gather(iref, [rows, kvec])
            wk = w0[k] if k < 16 else w1[k - 16]
            acc = acc + ucol * icol * wk
        z = acc + bias
        outv[pl.ds(g * 16, 16)] = 1.0 / (1.0 + jnp.exp(-z))
        return carry

    lax.fori_loop(0, NGROUP, group, None)

    pltpu.sync_copy(outv, out_hbm.at[pl.ds(base, BPW)])


@jax.jit
def kernel(x, user_table, item_table, W, b):
    wb = jnp.concatenate([W.reshape(K), jnp.pad(b, (0, 7))]).astype(jnp.float32)
    mesh = plsc.VectorSubcoreMesh(core_axis_name="c", subcore_axis_name="s")
    out = pl.kernel(
        _gmf_body,
        out_type=jax.ShapeDtypeStruct((B,), jnp.float32),
        mesh=mesh,
        compiler_params=pltpu.CompilerParams(
            needs_layout_passes=False, use_tc_tiling_on_sc=False),
        scratch_types=[
            pltpu.VMEM((2 * BPW,), jnp.int32),
            pltpu.VMEM((NCHUNK, 128), jnp.int32),
            pltpu.VMEM((NCHUNK, 128), jnp.int32),
            pltpu.VMEM((NCHUNK, 128), jnp.int32),
            pltpu.VMEM((NCHUNK, 128), jnp.int32),
            pltpu.VMEM((BPW, 16), jnp.float32),
            pltpu.VMEM((BPW, 16), jnp.float32),
            pltpu.VMEM((BPW, 16), jnp.float32),
            pltpu.VMEM((BPW, 16), jnp.float32),
            pltpu.VMEM((40,), jnp.float32),
            pltpu.VMEM((BPW,), jnp.float32),
            pltpu.SemaphoreType.DMA,
        ],
    )(x.astype(jnp.int32).reshape(2 * B),
      wb,
      user_table.reshape(-1, 16),
      item_table.reshape(-1, 16))
    return out.reshape(B, 1, 1)
